# baseline (speedup 1.0000x reference)
import functools

import jax
import jax.numpy as jnp
from jax import lax
from jax.experimental import pallas as pl
from jax.experimental.pallas import tpu as pltpu
from jax.experimental.pallas import tpu_sc as plsc

SIDE = 256
SCALE = 3.0
NPTS = 1048576
TSIZE = SIDE * SIDE * SIDE
NW = 32
PTS_PER_TILE = NPTS // NW
CHUNK = 128
NCHUNK = PTS_PER_TILE // CHUNK


def _prep_axis(v):
    iv = (v * (1.0 / SCALE) + 1.0) * (0.5 * SIDE) - 0.5
    iv = jnp.clip(iv, 0.0, float(SIDE - 1))
    i0 = iv.astype(jnp.int32)
    t = iv - i0.astype(jnp.float32)
    neg = t < 0.0
    i0 = jnp.where(neg, i0 - 1, i0)
    t = jnp.where(neg, t + 1.0, t)
    i1 = jnp.minimum(i0 + 1, SIDE - 1)
    return i0, i1, t


@functools.partial(
    pl.kernel,
    mesh=plsc.VectorSubcoreMesh(core_axis_name="c", subcore_axis_name="s"),
    out_type=jax.ShapeDtypeStruct((4 * NPTS,), jnp.float32),
    compiler_params=pltpu.CompilerParams(use_tc_tiling_on_sc=False),
    scratch_types=(
        [pltpu.VMEM((CHUNK,), jnp.float32) for _ in range(3)]
        + [pltpu.VMEM((CHUNK,), jnp.int32) for _ in range(32)]
        + [pltpu.VMEM((CHUNK,), jnp.float32) for _ in range(32)]
        + [
            pltpu.VMEM((64,), jnp.float32),
            pltpu.VMEM((CHUNK * 4,), jnp.float32),
            pltpu.SemaphoreType.DMA,
        ]
    ),
)
def _sc_sample(xs_hbm, ys_hbm, zs_hbm, table_hbm, bias_hbm, out_hbm, *scr):
    xs_v, ys_v, zs_v = scr[0:3]
    idx_refs = scr[3:35]
    val_refs = scr[35:67]
    bias_v, out_v, sem = scr[67:70]

    wid = lax.axis_index("s") * 2 + lax.axis_index("c")
    tile_base = wid * PTS_PER_TILE

    pltpu.sync_copy(bias_hbm, bias_v)

    def chunk_body(ci, carry):
        base = tile_base + ci * CHUNK
        pltpu.sync_copy(xs_hbm.at[pl.ds(base, CHUNK)], xs_v)
        pltpu.sync_copy(ys_hbm.at[pl.ds(base, CHUNK)], ys_v)
        pltpu.sync_copy(zs_hbm.at[pl.ds(base, CHUNK)], zs_v)

        for g in range(CHUNK // 16):
            sl = pl.ds(g * 16, 16)
            x0, x1, _ = _prep_axis(xs_v[sl])
            y0, y1, _ = _prep_axis(ys_v[sl])
            z0, z1, _ = _prep_axis(zs_v[sl])
            b00 = (z0 * SIDE + y0) * SIDE
            b01 = (z0 * SIDE + y1) * SIDE
            b10 = (z1 * SIDE + y0) * SIDE
            b11 = (z1 * SIDE + y1) * SIDE
            corner = (b00 + x0, b00 + x1, b01 + x0, b01 + x1,
                      b10 + x0, b10 + x1, b11 + x0, b11 + x1)
            for k in range(8):
                for c in range(4):
                    idx_refs[4 * k + c][sl] = corner[k] + (c * TSIZE)

        copies = []
        for i in range(32):
            copies.append(pltpu.async_copy(
                table_hbm.at[idx_refs[i]], val_refs[i], sem))
        for cp in copies:
            cp.wait()

        for g in range(CHUNK // 16):
            sl = pl.ds(g * 16, 16)
            _, _, tx = _prep_axis(xs_v[sl])
            _, _, ty = _prep_axis(ys_v[sl])
            _, _, tz = _prep_axis(zs_v[sl])
            wx0 = 1.0 - tx
            wy0 = 1.0 - ty
            wz0 = 1.0 - tz
            wzy = (wz0 * wy0, wz0 * ty, tz * wy0, tz * ty)
            w = [wzy[k >> 1] * (wx0 if (k & 1) == 0 else tx) for k in range(8)]
            for c in range(4):
                acc = bias_v[pl.ds(c * 16, 16)]
                for k in range(8):
                    acc = acc + val_refs[4 * k + c][sl] * w[k]
                out_v[pl.ds(c * CHUNK + g * 16, 16)] = acc

        for c in range(4):
            pltpu.sync_copy(out_v.at[pl.ds(c * CHUNK, CHUNK)],
                            out_hbm.at[pl.ds(c * NPTS + base, CHUNK)])
        return carry

    lax.fori_loop(0, NCHUNK, chunk_body, 0)


def kernel(positions, voxels, bias):
    pos_t = positions.reshape(NPTS, 3).T
    xs, ys, zs = pos_t[0], pos_t[1], pos_t[2]
    table = voxels.reshape(4 * TSIZE)
    bias64 = jnp.tile(bias.reshape(4, 1), (1, 16)).reshape(-1)
    out_planar = _sc_sample(xs, ys, zs, table, bias64)
    return out_planar.reshape(4, NPTS).T
